# baseline (speedup 1.0000x reference)
import functools

import jax
import jax.numpy as jnp
from jax import lax
from jax.experimental import pallas as pl
from jax.experimental.pallas import tpu as pltpu
from jax.experimental.pallas import tpu_sc as plsc

N_ATOMS_ = 3276800
_TABLE_N = 119
TABLE_PAD = 128

_info = plsc.get_sparse_core_info()
_NC, _NS, _L = _info.num_cores, _info.num_subcores, _info.num_lanes
_NW = _NC * _NS
_PER_W = N_ATOMS_ // _NW
_CHUNK = 6400
_N_CHUNKS = _PER_W // _CHUNK
_NBUF = 4


def _gather_kernel(table_hbm, an_hbm, out_hbm, table_v, *bufs_and_sems):
    idx_v = bufs_and_sems[:_NBUF]
    out_v = bufs_and_sems[_NBUF : 2 * _NBUF]
    in_sems = bufs_and_sems[2 * _NBUF : 3 * _NBUF]
    out_sems = bufs_and_sems[3 * _NBUF :]
    wid = lax.axis_index("s") * _NC + lax.axis_index("c")
    base_w = wid * _PER_W

    def in_copy(c, b):
        return pltpu.make_async_copy(
            an_hbm.at[pl.ds(base_w + c * _CHUNK, _CHUNK)], idx_v[b], in_sems[b]
        )

    def out_copy(c, b):
        return pltpu.make_async_copy(
            out_v[b], out_hbm.at[pl.ds(base_w + c * _CHUNK, _CHUNK)], out_sems[b]
        )

    for c in range(_NBUF):
        in_copy(c, c).start()
    pltpu.sync_copy(table_hbm, table_v.at[pl.ds(0, _TABLE_N)])

    for c in range(_N_CHUNKS):
        b = c % _NBUF
        in_copy(c, b).wait()
        if c >= _NBUF:
            out_copy(c - _NBUF, b).wait()

        @plsc.parallel_loop(0, _CHUNK, _L, unroll=16)
        def _(o):
            idx = idx_v[b][pl.ds(o, _L)]
            out_v[b][pl.ds(o, _L)] = plsc.load_gather(table_v, [idx])
        out_copy(c, b).start()
        if c + _NBUF < _N_CHUNKS:
            in_copy(c + _NBUF, b).start()

    for c in range(_N_CHUNKS - _NBUF, _N_CHUNKS):
        out_copy(c, c % _NBUF).wait()


@jax.jit
def _run(table_pad, atomic_numbers):
    k = functools.partial(
        pl.kernel,
        mesh=plsc.VectorSubcoreMesh(core_axis_name="c", subcore_axis_name="s"),
        out_type=jax.ShapeDtypeStruct((N_ATOMS_,), jnp.float32),
        scratch_types=[pltpu.VMEM((TABLE_PAD,), jnp.float32)]
        + [pltpu.VMEM((_CHUNK,), jnp.int32)] * _NBUF
        + [pltpu.VMEM((_CHUNK,), jnp.float32)] * _NBUF
        + [pltpu.SemaphoreType.DMA] * (2 * _NBUF),
        compiler_params=pltpu.CompilerParams(needs_layout_passes=False),
    )(_gather_kernel)
    return k(table_pad, atomic_numbers)


def kernel(atomic_numbers, atomic_subsystem_indices, ase_table):
    del atomic_subsystem_indices
    return _run(ase_table.astype(jnp.float32), atomic_numbers.astype(jnp.int32))


# speedup vs baseline: 1.0523x; 1.0523x over previous
import functools

import jax
import jax.numpy as jnp
from jax import lax
from jax.experimental import pallas as pl
from jax.experimental.pallas import tpu as pltpu
from jax.experimental.pallas import tpu_sc as plsc

N_ATOMS_ = 3276800
_TABLE_N = 119
TABLE_PAD = 128

_info = plsc.get_sparse_core_info()
_NC, _NS, _L = _info.num_cores, _info.num_subcores, _info.num_lanes
_NW = _NC * _NS
_PER_W = N_ATOMS_ // _NW
_CHUNK = 12800
_N_CHUNKS = _PER_W // _CHUNK
_NBUF = 4


def _gather_kernel(table_hbm, an_hbm, out_hbm, table_v, *bufs_and_sems):
    idx_v = bufs_and_sems[:_NBUF]
    out_v = bufs_and_sems[_NBUF : 2 * _NBUF]
    in_sems = bufs_and_sems[2 * _NBUF : 3 * _NBUF]
    out_sems = bufs_and_sems[3 * _NBUF :]
    wid = lax.axis_index("s") * _NC + lax.axis_index("c")
    base_w = wid * _PER_W

    def in_copy(c, b):
        return pltpu.make_async_copy(
            an_hbm.at[pl.ds(base_w + c * _CHUNK, _CHUNK)], idx_v[b], in_sems[b]
        )

    def out_copy(c, b):
        return pltpu.make_async_copy(
            out_v[b], out_hbm.at[pl.ds(base_w + c * _CHUNK, _CHUNK)], out_sems[b]
        )

    for c in range(_NBUF):
        in_copy(c, c).start()
    pltpu.sync_copy(table_hbm, table_v.at[pl.ds(0, _TABLE_N)])

    for c in range(_N_CHUNKS):
        b = c % _NBUF
        in_copy(c, b).wait()
        if c >= _NBUF:
            out_copy(c - _NBUF, b).wait()

        @plsc.parallel_loop(0, _CHUNK, _L, unroll=8)
        def _(o):
            idx = idx_v[b][pl.ds(o, _L)]
            out_v[b][pl.ds(o, _L)] = plsc.load_gather(table_v, [idx])
        out_copy(c, b).start()
        if c + _NBUF < _N_CHUNKS:
            in_copy(c + _NBUF, b).start()

    for c in range(_N_CHUNKS - _NBUF, _N_CHUNKS):
        out_copy(c, c % _NBUF).wait()


@jax.jit
def _run(table_pad, atomic_numbers):
    k = functools.partial(
        pl.kernel,
        mesh=plsc.VectorSubcoreMesh(core_axis_name="c", subcore_axis_name="s"),
        out_type=jax.ShapeDtypeStruct((N_ATOMS_,), jnp.float32),
        scratch_types=[pltpu.VMEM((TABLE_PAD,), jnp.float32)]
        + [pltpu.VMEM((_CHUNK,), jnp.int32)] * _NBUF
        + [pltpu.VMEM((_CHUNK,), jnp.float32)] * _NBUF
        + [pltpu.SemaphoreType.DMA] * (2 * _NBUF),
        compiler_params=pltpu.CompilerParams(needs_layout_passes=False),
    )(_gather_kernel)
    return k(table_pad, atomic_numbers)


def kernel(atomic_numbers, atomic_subsystem_indices, ase_table):
    del atomic_subsystem_indices
    return _run(ase_table.astype(jnp.float32), atomic_numbers.astype(jnp.int32))


# speedup vs baseline: 1.0540x; 1.0016x over previous
import functools

import jax
import jax.numpy as jnp
from jax import lax
from jax.experimental import pallas as pl
from jax.experimental.pallas import tpu as pltpu
from jax.experimental.pallas import tpu_sc as plsc

N_ATOMS_ = 3276800
_TABLE_N = 119
TABLE_PAD = 128

_info = plsc.get_sparse_core_info()
_NC, _NS, _L = _info.num_cores, _info.num_subcores, _info.num_lanes
_NW = _NC * _NS
_PER_W = N_ATOMS_ // _NW
_CHUNK = 20480
_N_CHUNKS = _PER_W // _CHUNK
_NBUF = 3


def _gather_kernel(table_hbm, an_hbm, out_hbm, table_v, *bufs_and_sems):
    idx_v = bufs_and_sems[:_NBUF]
    out_v = bufs_and_sems[_NBUF : 2 * _NBUF]
    in_sems = bufs_and_sems[2 * _NBUF : 3 * _NBUF]
    out_sems = bufs_and_sems[3 * _NBUF :]
    wid = lax.axis_index("s") * _NC + lax.axis_index("c")
    base_w = wid * _PER_W

    def in_copy(c, b):
        return pltpu.make_async_copy(
            an_hbm.at[pl.ds(base_w + c * _CHUNK, _CHUNK)], idx_v[b], in_sems[b]
        )

    def out_copy(c, b):
        return pltpu.make_async_copy(
            out_v[b], out_hbm.at[pl.ds(base_w + c * _CHUNK, _CHUNK)], out_sems[b]
        )

    for c in range(_NBUF):
        in_copy(c, c).start()
    pltpu.sync_copy(table_hbm, table_v.at[pl.ds(0, _TABLE_N)])

    for c in range(_N_CHUNKS):
        b = c % _NBUF
        in_copy(c, b).wait()
        if c >= _NBUF:
            out_copy(c - _NBUF, b).wait()

        @plsc.parallel_loop(0, _CHUNK, _L, unroll=8)
        def _(o):
            idx = idx_v[b][pl.ds(o, _L)]
            out_v[b][pl.ds(o, _L)] = plsc.load_gather(table_v, [idx])
        out_copy(c, b).start()
        if c + _NBUF < _N_CHUNKS:
            in_copy(c + _NBUF, b).start()

    for c in range(_N_CHUNKS - _NBUF, _N_CHUNKS):
        out_copy(c, c % _NBUF).wait()


@jax.jit
def _run(table_pad, atomic_numbers):
    k = functools.partial(
        pl.kernel,
        mesh=plsc.VectorSubcoreMesh(core_axis_name="c", subcore_axis_name="s"),
        out_type=jax.ShapeDtypeStruct((N_ATOMS_,), jnp.float32),
        scratch_types=[pltpu.VMEM((TABLE_PAD,), jnp.float32)]
        + [pltpu.VMEM((_CHUNK,), jnp.int32)] * _NBUF
        + [pltpu.VMEM((_CHUNK,), jnp.float32)] * _NBUF
        + [pltpu.SemaphoreType.DMA] * (2 * _NBUF),
        compiler_params=pltpu.CompilerParams(needs_layout_passes=False),
    )(_gather_kernel)
    return k(table_pad, atomic_numbers)


def kernel(atomic_numbers, atomic_subsystem_indices, ase_table):
    del atomic_subsystem_indices
    return _run(ase_table.astype(jnp.float32), atomic_numbers.astype(jnp.int32))
